# Initial kernel scaffold; baseline (speedup 1.0000x reference)
import jax
import jax.numpy as jnp
from jax.experimental import pallas as pl


def kernel(x_in, sigma, alpha, f_w, f_b, p_w, p_b, mlp_w1, mlp_b1, mlp_w2, mlp_b2):
    raise NotImplementedError("write your pallas kernel here")


# speedup vs baseline: 16.1756x; 16.1756x over previous
import jax
import jax.numpy as jnp
from jax.experimental import pallas as pl
from jax.experimental.pallas import tpu as pltpu

_HP = 16
_N = _HP * _HP
_K = 9
_NEG = -3.0e38


def _sigmoid(x):
    return 1.0 / (1.0 + jnp.exp(-x))


def _patch_body(scal_ref, x_ref, fw_ref, fb_ref, pw_ref, pb_ref, out_ref, d_scr):
    p = pl.program_id(0)
    sigma = scal_ref[0, 0]
    alpha = scal_ref[0, 1]

    @pl.when(p == 0)
    def _():
        ni = jax.lax.broadcasted_iota(jnp.int32, (_N, _N), 0)
        mi = jax.lax.broadcasted_iota(jnp.int32, (_N, _N), 1)
        dr = (ni // _HP) - (mi // _HP)
        dc = (ni % _HP) - (mi % _HP)
        d2 = (dr * dr + dc * dc).astype(jnp.float32)
        d_scr[...] = jnp.exp(d2 * (-1.0 / (2.0 * sigma * sigma)))

    D = d_scr[...]

    hi = jax.lax.Precision.DEFAULT
    xm = x_ref[0]
    ft = jax.lax.dot_general(fw_ref[...], xm, (((1,), (0,)), ((), ())),
                             precision=hi, preferred_element_type=jnp.float32)
    ft = ft + fb_ref[...]
    nsq = jnp.sum(ft * ft, axis=0)[None, :]
    inv = 1.0 / jnp.maximum(jnp.sqrt(nsq), 1e-8)
    ftn = ft * inv
    S = jax.lax.dot_general(ftn, ftn, (((0,), (0,)), ((), ())),
                            precision=hi, preferred_element_type=jnp.float32)
    comb = alpha * S + (1.0 - alpha) * D

    tot = scal_ref[0, 18]
    for i in range(4):
        h = S * scal_ref[0, 2 + 2 * i] + D * scal_ref[0, 3 + 2 * i] + scal_ref[0, 10 + i]
        h = h * _sigmoid(h)
        tot = tot + h * scal_ref[0, 14 + i]
    W = _sigmoid(tot)

    cur = comb
    M = jnp.zeros_like(comb)
    for _ in range(_K):
        m = jnp.max(cur, axis=1, keepdims=True)
        sel = cur >= m
        M = jnp.where(sel, 1.0, M)
        cur = jnp.where(sel, _NEG, cur)

    A = M * W
    wsum = jnp.sum(A, axis=1, keepdims=True)
    agg = jax.lax.dot_general(A, ft, (((1,), (1,)), ((), ())),
                              precision=hi, preferred_element_type=jnp.float32)
    agg = agg / (wsum + 1e-12)
    y = jax.lax.dot_general(pw_ref[...], agg, (((1,), (1,)), ((), ())),
                            precision=hi, preferred_element_type=jnp.float32)
    y = y + pb_ref[...]
    out_ref[...] = y[None]


@jax.jit
def kernel(x_in, sigma, alpha, f_w, f_b, p_w, p_b, mlp_w1, mlp_b1, mlp_w2, mlp_b2):
    B, C, H, Wd = x_in.shape
    ws = 7
    scal = jnp.concatenate([
        jnp.stack([sigma, alpha]),
        mlp_w1.reshape(-1), mlp_b1.reshape(-1),
        mlp_w2.reshape(-1), mlp_b2.reshape(-1),
    ]).reshape(1, 19).astype(jnp.float32)
    xp = x_in.reshape(C, ws, _HP, ws, _HP).transpose(1, 3, 0, 2, 4).reshape(ws * ws, C, _N)
    out = pl.pallas_call(
        _patch_body,
        grid=(ws * ws,),
        in_specs=[
            pl.BlockSpec((1, 19), lambda p: (0, 0), memory_space=pltpu.SMEM),
            pl.BlockSpec((1, C, _N), lambda p: (p, 0, 0)),
            pl.BlockSpec((32, C), lambda p: (0, 0)),
            pl.BlockSpec((32, 1), lambda p: (0, 0)),
            pl.BlockSpec((C, 32), lambda p: (0, 0)),
            pl.BlockSpec((C, 1), lambda p: (0, 0)),
        ],
        out_specs=pl.BlockSpec((1, C, _N), lambda p: (p, 0, 0)),
        out_shape=jax.ShapeDtypeStruct((ws * ws, C, _N), jnp.float32),
        scratch_shapes=[pltpu.VMEM((_N, _N), jnp.float32)],
    )(scal, xp, f_w, f_b.reshape(32, 1), p_w, p_b.reshape(128, 1))
    out = out.reshape(ws, ws, C, _HP, _HP).transpose(2, 0, 3, 1, 4).reshape(B, C, H * Wd)
    return out
